# Initial kernel scaffold; baseline (speedup 1.0000x reference)
import jax
import jax.numpy as jnp
from jax.experimental import pallas as pl


def kernel(SDist, srpe_weight):
    raise NotImplementedError("write your pallas kernel here")


# speedup vs baseline: 5.4796x; 5.4796x over previous
import functools

import jax
import jax.numpy as jnp
from jax import lax
from jax.experimental import pallas as pl
from jax.experimental.pallas import tpu as pltpu
from jax.experimental.pallas import tpu_sc as plsc

SEQ = 2048
D = 16
N = SEQ * SEQ
NC, NS = 2, 16
NW = NC * NS
ROWS_PER_W = N // NW
IPG = 128
CHUNK = 2048
GPC = CHUNK // IPG
NCHUNK = ROWS_PER_W // CHUNK

_mesh = plsc.VectorSubcoreMesh(core_axis_name="c", subcore_axis_name="s")


@functools.partial(
    pl.kernel,
    out_type=jax.ShapeDtypeStruct((N, D), jnp.float32),
    mesh=_mesh,
    compiler_params=pltpu.CompilerParams(use_tc_tiling_on_sc=False),
    scratch_types=[
        pltpu.VMEM((GPC, IPG), jnp.int32),
        pltpu.VMEM((CHUNK, D), jnp.float32),
        pltpu.SemaphoreType.DMA,
    ],
)
def _srpe_gather(idx_hbm, table_hbm, out_hbm, idx_v, rows_v, sem):
    wid = lax.axis_index("s") * NC + lax.axis_index("c")
    base = wid * ROWS_PER_W

    def chunk_body(g, carry):
        off = pl.multiple_of(base + g * CHUNK, CHUNK)
        pltpu.sync_copy(idx_hbm.at[pl.ds(pl.multiple_of(off // IPG, GPC), GPC), :], idx_v)
        for j in range(GPC):
            pltpu.async_copy(
                table_hbm.at[idx_v.at[j]],
                rows_v.at[pl.ds(j * IPG, IPG), :],
                sem,
            )
        for j in range(GPC):
            pltpu.make_async_copy(
                table_hbm.at[idx_v.at[j]],
                rows_v.at[pl.ds(j * IPG, IPG), :],
                sem,
            ).wait()
        pltpu.sync_copy(rows_v, out_hbm.at[pl.ds(off, CHUNK), :])
        return carry

    lax.fori_loop(0, NCHUNK, chunk_body, 0)


def kernel(SDist, srpe_weight):
    idx = SDist.reshape(N // IPG, IPG)
    out = _srpe_gather(idx, srpe_weight)
    return out.reshape(SEQ, SEQ, D)


# speedup vs baseline: 8.9868x; 1.6400x over previous
import functools

import jax
import jax.numpy as jnp
from jax import lax
from jax.experimental import pallas as pl
from jax.experimental.pallas import tpu as pltpu
from jax.experimental.pallas import tpu_sc as plsc

SEQ = 2048
D = 16
N = SEQ * SEQ
NC, NS = 2, 16
NW = NC * NS
ROWS_PER_W = N // NW
IPG = 128
CHUNK = 2048
GPC = CHUNK // IPG
NCHUNK = ROWS_PER_W // CHUNK

_mesh = plsc.VectorSubcoreMesh(core_axis_name="c", subcore_axis_name="s")


@functools.partial(
    pl.kernel,
    out_type=jax.ShapeDtypeStruct((N, D), jnp.float32),
    mesh=_mesh,
    compiler_params=pltpu.CompilerParams(use_tc_tiling_on_sc=False),
    scratch_types=[
        pltpu.VMEM((GPC, IPG), jnp.int32),
        pltpu.VMEM((CHUNK, D), jnp.float32),
        pltpu.VMEM_SHARED((129, D), jnp.float32),
        pltpu.SemaphoreType.DMA,
    ],
)
def _srpe_gather(idx_hbm, table_hbm, out_hbm, idx_v, rows_v, table_s, sem):
    wid = lax.axis_index("s") * NC + lax.axis_index("c")
    base = wid * ROWS_PER_W

    @pl.when(lax.axis_index("s") == 0)
    def _stage_table():
        pltpu.sync_copy(table_hbm, table_s)

    plsc.subcore_barrier()

    def chunk_body(g, carry):
        off = pl.multiple_of(base + g * CHUNK, CHUNK)
        pltpu.sync_copy(idx_hbm.at[pl.ds(pl.multiple_of(off // IPG, GPC), GPC), :], idx_v)
        for j in range(GPC):
            pltpu.async_copy(
                table_s.at[idx_v.at[j]],
                rows_v.at[pl.ds(j * IPG, IPG), :],
                sem,
            )
        for j in range(GPC):
            pltpu.make_async_copy(
                table_s.at[idx_v.at[j]],
                rows_v.at[pl.ds(j * IPG, IPG), :],
                sem,
            ).wait()
        pltpu.sync_copy(rows_v, out_hbm.at[pl.ds(off, CHUNK), :])
        return carry

    lax.fori_loop(0, NCHUNK, chunk_body, 0)


def kernel(SDist, srpe_weight):
    idx = SDist.reshape(N // IPG, IPG)
    out = _srpe_gather(idx, srpe_weight)
    return out.reshape(SEQ, SEQ, D)


# speedup vs baseline: 9.5711x; 1.0650x over previous
import functools

import jax
import jax.numpy as jnp
from jax import lax
from jax.experimental import pallas as pl
from jax.experimental.pallas import tpu as pltpu
from jax.experimental.pallas import tpu_sc as plsc

SEQ = 2048
D = 16
N = SEQ * SEQ
NC, NS = 2, 16
NW = NC * NS
ROWS_PER_W = N // NW
IPG = 128
CHUNK = 2048
GPC = CHUNK // IPG
NCHUNK = ROWS_PER_W // CHUNK

_mesh = plsc.VectorSubcoreMesh(core_axis_name="c", subcore_axis_name="s")


@functools.partial(
    pl.kernel,
    out_type=jax.ShapeDtypeStruct((N, D), jnp.float32),
    mesh=_mesh,
    compiler_params=pltpu.CompilerParams(use_tc_tiling_on_sc=False),
    scratch_types=[
        pltpu.VMEM((2, GPC, IPG), jnp.int32),
        pltpu.VMEM((2, CHUNK, D), jnp.float32),
        pltpu.VMEM_SHARED((129, D), jnp.float32),
        pltpu.SemaphoreType.DMA,
        pltpu.SemaphoreType.DMA,
        pltpu.SemaphoreType.DMA,
        pltpu.SemaphoreType.DMA,
        pltpu.SemaphoreType.DMA,
    ],
)
def _srpe_gather(idx_hbm, table_hbm, out_hbm, idx_v, rows_v, table_s,
                 sem_i0, sem_i1, sem_g, sem_o0, sem_o1):
    wid = lax.axis_index("s") * NC + lax.axis_index("c")
    base = wid * ROWS_PER_W

    @pl.when(lax.axis_index("s") == 0)
    def _stage_table():
        pltpu.sync_copy(table_hbm, table_s)

    plsc.subcore_barrier()

    sem_i = (sem_i0, sem_i1)
    sem_o = (sem_o0, sem_o1)

    def idx_src(gg):
        off = pl.multiple_of(base + gg * CHUNK, CHUNK)
        return idx_hbm.at[pl.ds(pl.multiple_of(off // IPG, GPC), GPC), :]

    def out_dst(gg):
        off = pl.multiple_of(base + gg * CHUNK, CHUNK)
        return out_hbm.at[pl.ds(off, CHUNK), :]

    pltpu.async_copy(idx_src(0), idx_v.at[0], sem_i[0])

    def outer_body(g, carry):
        for b in range(2):
            gg = g + b
            @pl.when(gg + 1 < NCHUNK)
            def _prefetch():
                pltpu.async_copy(idx_src(gg + 1), idx_v.at[1 - b], sem_i[1 - b])

            pltpu.make_async_copy(idx_src(gg), idx_v.at[b], sem_i[b]).wait()

            @pl.when(gg >= 2)
            def _drain_prev():
                pltpu.make_async_copy(rows_v.at[b], out_dst(gg), sem_o[b]).wait()

            for j in range(GPC):
                pltpu.async_copy(
                    table_s.at[idx_v.at[b].at[j]],
                    rows_v.at[b].at[pl.ds(j * IPG, IPG), :],
                    sem_g,
                )
            for j in range(GPC):
                pltpu.make_async_copy(
                    table_s.at[idx_v.at[b].at[j]],
                    rows_v.at[b].at[pl.ds(j * IPG, IPG), :],
                    sem_g,
                ).wait()

            pltpu.async_copy(rows_v.at[b], out_dst(gg), sem_o[b])
        return carry

    lax.fori_loop(0, NCHUNK // 2, lambda i, c: outer_body(i * 2, c), 0)

    pltpu.make_async_copy(rows_v.at[0], out_dst(NCHUNK - 2), sem_o[0]).wait()
    pltpu.make_async_copy(rows_v.at[1], out_dst(NCHUNK - 1), sem_o[1]).wait()


def kernel(SDist, srpe_weight):
    idx = SDist.reshape(N // IPG, IPG)
    out = _srpe_gather(idx, srpe_weight)
    return out.reshape(SEQ, SEQ, D)


# speedup vs baseline: 26.2517x; 2.7428x over previous
import functools

import jax
import jax.numpy as jnp
from jax import lax
from jax.experimental import pallas as pl
from jax.experimental.pallas import tpu as pltpu
from jax.experimental.pallas import tpu_sc as plsc

SEQ = 2048
D = 16
N = SEQ * SEQ
NC, NS = 2, 16
NW = NC * NS
L = 16
ROW_OUT = 2 * 16 * 8 * 128
IBLK = 16 * 8 * 128
BLKS_PER_W = (SEQ // 8) // NW

_mesh = plsc.VectorSubcoreMesh(core_axis_name="c", subcore_axis_name="s")


@functools.partial(
    pl.kernel,
    out_type=jax.ShapeDtypeStruct((SEQ * ROW_OUT,), jnp.float32),
    mesh=_mesh,
    compiler_params=pltpu.CompilerParams(
        use_tc_tiling_on_sc=False, needs_layout_passes=False),
    scratch_types=[
        pltpu.VMEM((2, IBLK), jnp.int32),
        pltpu.VMEM((2, ROW_OUT), jnp.float32),
        pltpu.VMEM((D, 129), jnp.float32),
        pltpu.SemaphoreType.DMA,
        pltpu.SemaphoreType.DMA,
        pltpu.SemaphoreType.DMA,
        pltpu.SemaphoreType.DMA,
    ],
)
def _srpe_gather(idx_hbm, tabt_hbm, out_hbm, idx_v, row_v, tabt_v,
                 sem_i0, sem_i1, sem_o0, sem_o1):
    wid = lax.axis_index("s") * NC + lax.axis_index("c")
    first_blk = wid * BLKS_PER_W

    pltpu.sync_copy(tabt_hbm, tabt_v)

    sem_i = (sem_i0, sem_i1)
    sem_o = (sem_o0, sem_o1)

    def idx_src(blk):
        off = pl.multiple_of(blk * IBLK, IBLK)
        return idx_hbm.at[pl.ds(off, IBLK)]

    def out_dst(row):
        off = pl.multiple_of(row * ROW_OUT, ROW_OUT)
        return out_hbm.at[pl.ds(off, ROW_OUT)]

    def gather_row(ib, i_in, rb):
        ibuf = idx_v.at[ib]
        obuf = row_v.at[rb]

        def jblk_body(j_blk, carry):
            i_off = j_blk * 1024 + i_in * 128
            o_off = j_blk * 1024
            for g in range(8):
                idx_vec = ibuf[pl.ds(i_off + g * L, L)]
                for d in range(D):
                    vals = plsc.load_gather(tabt_v.at[d], [idx_vec])
                    o = (d // 8) * 16384 + (d % 8) * 128 + g * L
                    obuf[pl.ds(o_off + o, L)] = vals
            return carry

        lax.fori_loop(0, 16, jblk_body, 0)

    pltpu.async_copy(idx_src(first_blk), idx_v.at[0], sem_i[0])

    def blk_body(b2, carry):
        for ib in range(2):
            b = b2 * 2 + ib
            blk = first_blk + b

            @pl.when(b + 1 < BLKS_PER_W)
            def _prefetch():
                pltpu.async_copy(
                    idx_src(blk + 1), idx_v.at[1 - ib], sem_i[1 - ib])

            pltpu.make_async_copy(idx_src(blk), idx_v.at[ib], sem_i[ib]).wait()

            def iin_body(ii, carry2, b=b, blk=blk, ib=ib):
                for rb in range(2):
                    i_in = ii * 2 + rb
                    row = blk * 8 + i_in

                    @pl.when(b * 8 + i_in >= 2)
                    def _drain():
                        pltpu.make_async_copy(
                            row_v.at[rb], out_dst(row), sem_o[rb]).wait()

                    gather_row(ib, i_in, rb)
                    pltpu.async_copy(row_v.at[rb], out_dst(row), sem_o[rb])
                return carry2

            lax.fori_loop(0, 4, iin_body, 0)
        return carry

    lax.fori_loop(0, BLKS_PER_W // 2, blk_body, 0)

    last = (first_blk + BLKS_PER_W) * 8
    pltpu.make_async_copy(row_v.at[0], out_dst(last - 2), sem_o[0]).wait()
    pltpu.make_async_copy(row_v.at[1], out_dst(last - 1), sem_o[1]).wait()


def kernel(SDist, srpe_weight):
    idx = SDist.reshape(SEQ // 8, 8, SEQ // 128, 128).transpose(0, 2, 1, 3)
    idx = idx.reshape(N)
    tabt = srpe_weight.T
    flat = _srpe_gather(idx, tabt)
    out = flat.reshape(SEQ, 2, 16, 8, 128).transpose(0, 2, 4, 1, 3)
    return out.reshape(SEQ, SEQ, D)


# speedup vs baseline: 140.0254x; 5.3339x over previous
import functools

import jax
import jax.numpy as jnp
from jax import lax
from jax.experimental import pallas as pl
from jax.experimental.pallas import tpu as pltpu
from jax.experimental.pallas import tpu_sc as plsc

SEQ = 2048
D = 16
N = SEQ * SEQ
NC, NS = 2, 16
NW = NC * NS
L = 16
ROW_OUT = 2 * 16 * 8 * 128
IBLK = 16 * 8 * 128
BLKS_PER_W = (SEQ // 8) // NW

_mesh = plsc.VectorSubcoreMesh(core_axis_name="c", subcore_axis_name="s")


@functools.partial(
    pl.kernel,
    out_type=jax.ShapeDtypeStruct((SEQ * ROW_OUT,), jnp.float32),
    mesh=_mesh,
    compiler_params=pltpu.CompilerParams(
        use_tc_tiling_on_sc=False, needs_layout_passes=False),
    scratch_types=[
        pltpu.VMEM((2, IBLK), jnp.int32),
        pltpu.VMEM((2, ROW_OUT), jnp.float32),
        pltpu.VMEM((D, 129), jnp.float32),
        pltpu.SemaphoreType.DMA,
        pltpu.SemaphoreType.DMA,
        pltpu.SemaphoreType.DMA,
        pltpu.SemaphoreType.DMA,
    ],
)
def _srpe_gather(idx_hbm, tabt_hbm, out_hbm, idx_v, row_v, tabt_v,
                 sem_i0, sem_i1, sem_o0, sem_o1):
    wid = lax.axis_index("s") * NC + lax.axis_index("c")
    first_blk = wid * BLKS_PER_W

    pltpu.sync_copy(tabt_hbm, tabt_v)

    sem_i = (sem_i0, sem_i1)
    sem_o = (sem_o0, sem_o1)

    def idx_src(blk):
        off = pl.multiple_of(blk * IBLK, IBLK)
        return idx_hbm.at[pl.ds(off, IBLK)]

    def out_dst(row):
        off = pl.multiple_of(row * ROW_OUT, ROW_OUT)
        return out_hbm.at[pl.ds(off, ROW_OUT)]

    def gather_row(ib, i_in, rb):
        ibuf = idx_v.at[ib]
        obuf = row_v.at[rb]

        @functools.partial(plsc.parallel_loop, 0, 128, unroll=4)
        def _group(k):
            jb = k >> 3
            g = k & 7
            io = jb * 1024 + i_in * 128 + g * L
            oo = jb * 1024 + g * L
            idx_vec = ibuf[pl.ds(io, L)]
            for d in range(D):
                vals = plsc.load_gather(tabt_v.at[d], [idx_vec])
                obuf[pl.ds(oo + (d // 8) * 16384 + (d % 8) * 128, L)] = vals

    pltpu.async_copy(idx_src(first_blk), idx_v.at[0], sem_i[0])

    def blk_body(b2, carry):
        for ib in range(2):
            b = b2 * 2 + ib
            blk = first_blk + b

            @pl.when(b + 1 < BLKS_PER_W)
            def _prefetch():
                pltpu.async_copy(
                    idx_src(blk + 1), idx_v.at[1 - ib], sem_i[1 - ib])

            pltpu.make_async_copy(idx_src(blk), idx_v.at[ib], sem_i[ib]).wait()

            def iin_body(ii, carry2, b=b, blk=blk, ib=ib):
                for rb in range(2):
                    i_in = ii * 2 + rb
                    row = blk * 8 + i_in

                    @pl.when(b * 8 + i_in >= 2)
                    def _drain():
                        pltpu.make_async_copy(
                            row_v.at[rb], out_dst(row), sem_o[rb]).wait()

                    gather_row(ib, i_in, rb)
                    pltpu.async_copy(row_v.at[rb], out_dst(row), sem_o[rb])
                return carry2

            lax.fori_loop(0, 4, iin_body, 0)
        return carry

    lax.fori_loop(0, BLKS_PER_W // 2, blk_body, 0)

    last = (first_blk + BLKS_PER_W) * 8
    pltpu.make_async_copy(row_v.at[0], out_dst(last - 2), sem_o[0]).wait()
    pltpu.make_async_copy(row_v.at[1], out_dst(last - 1), sem_o[1]).wait()


def kernel(SDist, srpe_weight):
    idx = SDist.reshape(SEQ // 8, 8, SEQ // 128, 128).transpose(0, 2, 1, 3)
    idx = idx.reshape(N)
    tabt = srpe_weight.T
    flat = _srpe_gather(idx, tabt)
    out = flat.reshape(SEQ, 2, 16, 8, 128).transpose(0, 2, 4, 1, 3)
    return out.reshape(SEQ, SEQ, D)
